# Initial kernel scaffold; baseline (speedup 1.0000x reference)
import jax
import jax.numpy as jnp
from jax.experimental import pallas as pl


def kernel(x, w1, b1, w2, b2, w3, b3, fw1, fb1, fw2, fb2):
    raise NotImplementedError("write your pallas kernel here")


# speedup vs baseline: 60.4404x; 60.4404x over previous
import numpy as np

import jax
import jax.numpy as jnp
from jax.experimental import pallas as pl
from jax.experimental.pallas import tpu as pltpu

_B = 128


def _np_w1_plan():
    r = np.arange(160)
    i, v = r // 32, r % 32
    c = np.arange(256)
    p, rem = c // 128, c % 128
    pw, co = rem // 6, rem % 6
    j = v[:, None] - (2 * pw + p)[None, :]
    valid = (rem[None, :] < 84) & (j >= 0) & (j <= 4)
    tap = i[:, None] * 5 + np.clip(j, 0, 4)
    co2 = np.broadcast_to(co[None, :], tap.shape)
    return tap, co2, valid


def _np_w2_plan():
    r = np.arange(420)
    i, rr = r // 84, r % 84
    wi, ci = rr // 6, rr % 6
    c = np.arange(256)
    p, rem = c // 128, c % 128
    pw, co = rem // 16, rem % 16
    j = wi[:, None] - (2 * pw + p)[None, :]
    valid = (rem[None, :] < 80) & (j >= 0) & (j <= 4)
    tap = i[:, None] * 5 + np.clip(j, 0, 4)
    ci2 = np.broadcast_to(ci[:, None], tap.shape)
    co2 = np.broadcast_to(co[None, :], tap.shape)
    return tap, ci2, co2, valid


_W1_TAP, _W1_CO, _W1_OK = _np_w1_plan()
_W2_TAP, _W2_CI, _W2_CO, _W2_OK = _np_w2_plan()
_B1_CO = (np.arange(128) % 6).astype(np.int32)
_B1_OK = np.arange(128) < 84
_B2_CO = (np.arange(128) % 16).astype(np.int32)
_B2_OK = np.arange(128) < 80


def _lenet_body(x_ref, w1_ref, b1_ref, w2_ref, b2_ref, w3_ref, b3_ref,
                fw1_ref, fb1_ref, fw2_ref, fb2_ref, o_ref,
                cat1, a1, cat2, a2, cat3):
    f32, bf16 = jnp.float32, jnp.bfloat16
    B = _B

    for phi in range(4):
        for i in range(5):
            sp = (phi + i) & 3
            s = (phi + i) >> 2
            cat1[:, phi * 8:phi * 8 + 8, 32 * i:32 * i + 32] = \
                x_ref[sp, 0, :, s:s + 8, :]

    a1[...] = jnp.dot(cat1[...].reshape(B * 32, 160), w1_ref[...],
                      preferred_element_type=f32).reshape(B, 32, 256)

    b1v = b1_ref[...][None]
    me = jnp.maximum(a1[:, 0:8, :], a1[:, 8:16, :])
    mo = jnp.maximum(a1[:, 16:24, :], a1[:, 24:32, :])
    p1e = jnp.maximum(jnp.maximum(me[:, :, 0:128], me[:, :, 128:256]) + b1v,
                      0.0).astype(bf16)
    p1o = jnp.maximum(jnp.maximum(mo[:, :, 0:128], mo[:, :, 128:256]) + b1v,
                      0.0).astype(bf16)

    for q in range(2):
        for i in range(5):
            src = p1e if ((q + i) & 1) == 0 else p1o
            off = (q + i) >> 1
            cat2[:, q * 8:q * 8 + 5, 84 * i:84 * i + 84] = \
                src[:, off:off + 5, 0:84]

    a2[...] = jnp.dot(cat2[...].reshape(B * 16, 420), w2_ref[...],
                      preferred_element_type=f32).reshape(B, 16, 256)

    b2v = b2_ref[...][None]
    me2 = jnp.maximum(a2[:, 0:5, :], a2[:, 8:13, :])
    p2 = jnp.maximum(jnp.maximum(me2[:, :, 0:128], me2[:, :, 128:256]) + b2v,
                     0.0)

    for i in range(5):
        cat3[:, 80 * i:80 * i + 80] = p2[:, i, 0:80].astype(bf16)

    h3 = jnp.dot(cat3[...], w3_ref[...], preferred_element_type=f32)
    h3 = jnp.maximum(h3 + b3_ref[...], 0.0)

    h4 = jnp.dot(h3.astype(bf16), fw1_ref[...], preferred_element_type=f32)
    h4 = jnp.maximum(h4 + fb1_ref[...], 0.0)
    out = jnp.dot(h4.astype(bf16), fw2_ref[...], preferred_element_type=f32)
    o_ref[0] = out + fb2_ref[...]


def kernel(x, w1, b1, w2, b2, w3, b3, fw1, fb1, fw2, fb2):
    N = x.shape[0]
    B = _B
    G = N // B

    x2 = x.reshape(N, 28, 28)
    xp = jnp.pad(x2, ((0, 0), (2, 6), (2, 2))).astype(jnp.bfloat16)
    xq = xp.reshape(N, 9, 4, 32)
    X4 = jnp.transpose(xq, (2, 0, 1, 3)).reshape(4, G, B, 9, 32)

    W1 = jnp.where(_W1_OK, w1[_W1_TAP, 0, _W1_CO], jnp.bfloat16(0))
    W2 = jnp.where(_W2_OK, w2[_W2_TAP, _W2_CI, _W2_CO], jnp.bfloat16(0))
    W3 = w3[:, :16, :].reshape(400, 128)
    b1p = jnp.where(_B1_OK, b1[0, _B1_CO], 0.0).reshape(1, 128)
    b2p = jnp.where(_B2_OK, b2[0, _B2_CO], 0.0).reshape(1, 128)

    out = pl.pallas_call(
        _lenet_body,
        out_shape=jax.ShapeDtypeStruct((G, B, 128), jnp.float32),
        grid=(G,),
        in_specs=[
            pl.BlockSpec((4, 1, B, 9, 32), lambda n: (0, n, 0, 0, 0)),
            pl.BlockSpec((160, 256), lambda n: (0, 0)),
            pl.BlockSpec((1, 128), lambda n: (0, 0)),
            pl.BlockSpec((420, 256), lambda n: (0, 0)),
            pl.BlockSpec((1, 128), lambda n: (0, 0)),
            pl.BlockSpec((400, 128), lambda n: (0, 0)),
            pl.BlockSpec((1, 128), lambda n: (0, 0)),
            pl.BlockSpec((128, 128), lambda n: (0, 0)),
            pl.BlockSpec((1, 128), lambda n: (0, 0)),
            pl.BlockSpec((128, 128), lambda n: (0, 0)),
            pl.BlockSpec((1, 128), lambda n: (0, 0)),
        ],
        out_specs=pl.BlockSpec((1, B, 128), lambda n: (n, 0, 0)),
        scratch_shapes=[
            pltpu.VMEM((B, 32, 160), jnp.bfloat16),
            pltpu.VMEM((B, 32, 256), jnp.float32),
            pltpu.VMEM((B, 16, 420), jnp.bfloat16),
            pltpu.VMEM((B, 16, 256), jnp.float32),
            pltpu.VMEM((B, 400), jnp.bfloat16),
        ],
        compiler_params=pltpu.CompilerParams(
            dimension_semantics=("parallel",),
            vmem_limit_bytes=64 * 1024 * 1024),
    )(X4, W1, b1p, W2, b2p, W3, b3, fw1, fb1, fw2, fb2)

    return out.reshape(N, 128)[:, :10]


# speedup vs baseline: 68.2783x; 1.1297x over previous
import jax
import jax.numpy as jnp
from jax.experimental import pallas as pl
from jax.experimental.pallas import tpu as pltpu

_B = 128


def _pack_w1(w1):
    w1s = w1[:, 0, :6].reshape(5, 5, 6)
    q = jnp.pad(w1s, ((0, 0), (27, 27), (0, 0)))
    rows = [q[:, 27 - 2 * pw - p:59 - 2 * pw - p, :]
            for p in range(2) for pw in range(14)]
    o = jnp.stack(rows, 0).reshape(2, 14, 5, 32, 6)
    o = jnp.transpose(o, (2, 3, 0, 1, 4)).reshape(160, 2, 84)
    return jnp.pad(o, ((0, 0), (0, 0), (0, 44))).reshape(160, 256)


def _pack_w2(w2):
    w2s = jnp.transpose(w2[:, :6, :16].reshape(5, 5, 6, 16), (0, 1, 3, 2))
    q = jnp.pad(w2s, ((0, 0), (9, 9), (0, 0), (0, 0)))
    rows = [q[:, 9 - 2 * pw - p:23 - 2 * pw - p, :, :]
            for p in range(2) for pw in range(5)]
    o = jnp.stack(rows, 0).reshape(2, 5, 5, 14, 16, 6)
    o = jnp.transpose(o, (2, 3, 5, 0, 1, 4)).reshape(420, 2, 80)
    return jnp.pad(o, ((0, 0), (0, 0), (0, 48))).reshape(420, 256)


def _lenet_body(x_ref, w1_ref, b1_ref, w2_ref, b2_ref, w3_ref, b3_ref,
                fw1_ref, fb1_ref, fw2_ref, fb2_ref, o_ref,
                cat1, a1, cat2, a2, cat3):
    f32, bf16 = jnp.float32, jnp.bfloat16
    B = _B

    for phi in range(4):
        for i in range(5):
            sp = (phi + i) & 3
            s = (phi + i) >> 2
            cat1[:, phi * 8:phi * 8 + 8, 32 * i:32 * i + 32] = \
                x_ref[sp, 0, :, s:s + 8, :]

    a1[...] = jnp.dot(cat1[...].reshape(B * 32, 160), w1_ref[...],
                      preferred_element_type=f32).reshape(B, 32, 256)

    b1v = b1_ref[...][None]
    me = jnp.maximum(a1[:, 0:8, :], a1[:, 8:16, :])
    mo = jnp.maximum(a1[:, 16:24, :], a1[:, 24:32, :])
    p1e = jnp.maximum(jnp.maximum(me[:, :, 0:128], me[:, :, 128:256]) + b1v,
                      0.0).astype(bf16)
    p1o = jnp.maximum(jnp.maximum(mo[:, :, 0:128], mo[:, :, 128:256]) + b1v,
                      0.0).astype(bf16)

    for q in range(2):
        for i in range(5):
            src = p1e if ((q + i) & 1) == 0 else p1o
            off = (q + i) >> 1
            cat2[:, q * 8:q * 8 + 5, 84 * i:84 * i + 84] = \
                src[:, off:off + 5, 0:84]

    a2[...] = jnp.dot(cat2[...].reshape(B * 16, 420), w2_ref[...],
                      preferred_element_type=f32).reshape(B, 16, 256)

    b2v = b2_ref[...][None]
    me2 = jnp.maximum(a2[:, 0:5, :], a2[:, 8:13, :])
    p2 = jnp.maximum(jnp.maximum(me2[:, :, 0:128], me2[:, :, 128:256]) + b2v,
                     0.0)

    for i in range(5):
        cat3[:, 80 * i:80 * i + 80] = p2[:, i, 0:80].astype(bf16)

    h3 = jnp.dot(cat3[...], w3_ref[...], preferred_element_type=f32)
    h3 = jnp.maximum(h3 + b3_ref[...], 0.0)

    h4 = jnp.dot(h3.astype(bf16), fw1_ref[...], preferred_element_type=f32)
    h4 = jnp.maximum(h4 + fb1_ref[...], 0.0)
    out = jnp.dot(h4.astype(bf16), fw2_ref[...], preferred_element_type=f32)
    o_ref[0] = out + fb2_ref[...]


def kernel(x, w1, b1, w2, b2, w3, b3, fw1, fb1, fw2, fb2):
    N = x.shape[0]
    B = _B
    G = N // B

    x2 = x.reshape(N, 28, 28)
    xp = jnp.pad(x2, ((0, 0), (2, 6), (2, 2))).astype(jnp.bfloat16)
    xq = xp.reshape(N, 9, 4, 32)
    X4 = jnp.transpose(xq, (2, 0, 1, 3)).reshape(4, G, B, 9, 32)

    W1 = _pack_w1(w1)
    W2 = _pack_w2(w2)
    W3 = w3[:, :16, :].reshape(400, 128)
    b1p = jnp.tile(b1[:, :6], (1, 22))[:, :128]
    b2p = jnp.tile(b2[:, :16], (1, 8))

    out = pl.pallas_call(
        _lenet_body,
        out_shape=jax.ShapeDtypeStruct((G, B, 128), jnp.float32),
        grid=(G,),
        in_specs=[
            pl.BlockSpec((4, 1, B, 9, 32), lambda n: (0, n, 0, 0, 0)),
            pl.BlockSpec((160, 256), lambda n: (0, 0)),
            pl.BlockSpec((1, 128), lambda n: (0, 0)),
            pl.BlockSpec((420, 256), lambda n: (0, 0)),
            pl.BlockSpec((1, 128), lambda n: (0, 0)),
            pl.BlockSpec((400, 128), lambda n: (0, 0)),
            pl.BlockSpec((1, 128), lambda n: (0, 0)),
            pl.BlockSpec((128, 128), lambda n: (0, 0)),
            pl.BlockSpec((1, 128), lambda n: (0, 0)),
            pl.BlockSpec((128, 128), lambda n: (0, 0)),
            pl.BlockSpec((1, 128), lambda n: (0, 0)),
        ],
        out_specs=pl.BlockSpec((1, B, 128), lambda n: (n, 0, 0)),
        scratch_shapes=[
            pltpu.VMEM((B, 32, 160), jnp.bfloat16),
            pltpu.VMEM((B, 32, 256), jnp.float32),
            pltpu.VMEM((B, 16, 420), jnp.bfloat16),
            pltpu.VMEM((B, 16, 256), jnp.float32),
            pltpu.VMEM((B, 400), jnp.bfloat16),
        ],
        compiler_params=pltpu.CompilerParams(
            dimension_semantics=("parallel",),
            vmem_limit_bytes=64 * 1024 * 1024),
    )(X4, W1, b1p, W2, b2p, W3, b3, fw1, fb1, fw2, fb2)

    return out.reshape(N, 128)[:, :10]
